# baseline (speedup 1.0000x reference)
import functools

import jax
import jax.numpy as jnp
from jax import lax
from jax.experimental import pallas as pl
from jax.experimental.pallas import tpu as pltpu
from jax.experimental.pallas import tpu_sc as plsc

B = 4096
L = 50
E = 64
EP = 128
H = 512

P = 10
LP = L // P
PW = P * EP
K = L * EP

NCH = 4
BC = B // NCH

NC, NS = 2, 16
NW = NC * NS
B_PER_W = BC // NW
CHUNK = 16
ROUNDS = (B_PER_W // CHUNK) * LP
IDX_PER_W = L * B_PER_W

BM = 256


def _sc_gather(emb_pad, idx_c):
    mesh = plsc.VectorSubcoreMesh(core_axis_name="c", subcore_axis_name="s")

    @functools.partial(
        pl.kernel,
        mesh=mesh,
        out_type=jax.ShapeDtypeStruct((BC, K), jnp.float32),
        scratch_types=[
            pltpu.VMEM((IDX_PER_W,), jnp.int32),
            pltpu.VMEM((CHUNK, PW), jnp.float32),
            pltpu.VMEM((CHUNK, PW), jnp.float32),
            pltpu.SemaphoreType.DMA,
            pltpu.SemaphoreType.DMA,
            pltpu.SemaphoreType.DMA,
            pltpu.SemaphoreType.DMA,
        ],
    )
    def k(emb_hbm, idx_hbm, out_hbm, idx_v, buf0, buf1, g0, g1, w0, w1):
        wid = lax.axis_index("s") * NC + lax.axis_index("c")
        b0 = wid * B_PER_W
        pltpu.sync_copy(idx_hbm.at[pl.ds(wid * IDX_PER_W, IDX_PER_W)], idx_v)

        def fire(r, buf, sem):
            base = r // 2 * (2 * CHUNK) + (r % 2) * CHUNK
            for p in range(P):
                pltpu.async_copy(
                    emb_hbm.at[idx_v.at[pl.ds(p * (2 * CHUNK * LP) + base, CHUNK)]],
                    buf.at[:, pl.ds(p * EP, EP)],
                    sem,
                )

        def out_slice(r):
            return out_hbm.at[
                pl.ds(b0 + (r % 2) * CHUNK, CHUNK), pl.ds(r // 2 * PW, PW)
            ]

        def drain(buf, sem):
            pltpu.make_async_copy(
                out_hbm.at[pl.ds(0, CHUNK), pl.ds(0, PW)], buf, sem
            ).wait()

        fire(0, buf0, g0)
        fire(1, buf1, g1)

        @pl.loop(0, ROUNDS // 2)
        def _(g):
            r0 = 2 * g
            r1 = r0 + 1
            drain(buf0, g0)
            pltpu.async_copy(buf0, out_slice(r0), w0)
            drain(buf1, g1)
            pltpu.async_copy(buf1, out_slice(r1), w1)

            @pl.when(r0 + 2 < ROUNDS)
            def _():
                drain(buf0, w0)
                fire(r0 + 2, buf0, g0)

            @pl.when(r1 + 2 < ROUNDS)
            def _():
                drain(buf1, w1)
                fire(r1 + 2, buf1, g1)

        drain(buf0, w0)
        drain(buf1, w1)

    return k(emb_pad, idx_c)


def _mm_body(x_ref, w_ref, b_ref, o_ref):
    acc = jnp.dot(
        x_ref[...].astype(jnp.bfloat16),
        w_ref[...],
        preferred_element_type=jnp.float32,
    )
    o_ref[...] = jnp.maximum(acc + b_ref[...], 0.0)


def _tc_matmul(g, Wp, b):
    return pl.pallas_call(
        _mm_body,
        grid=(BC // BM,),
        in_specs=[
            pl.BlockSpec((BM, K), lambda i: (i, 0)),
            pl.BlockSpec((K, H), lambda i: (0, 0)),
            pl.BlockSpec((1, H), lambda i: (0, 0)),
        ],
        out_specs=pl.BlockSpec((BM, H), lambda i: (i, 0)),
        out_shape=jax.ShapeDtypeStruct((BC, H), jnp.float32),
    )(g, Wp, b.reshape(1, H))


def kernel(x, emb, W, b):
    idx_all = (
        x.astype(jnp.int32)
        .reshape(NCH, NW, 2, CHUNK, LP, P)
        .transpose(0, 1, 5, 4, 2, 3)
        .reshape(NCH, NW * IDX_PER_W)
    )
    emb_pad = jnp.pad(emb, ((0, 0), (0, EP - E)))
    Wp = jnp.concatenate(
        [
            W.reshape(L, E, H).astype(jnp.bfloat16),
            jnp.zeros((L, EP - E, H), jnp.bfloat16),
        ],
        axis=1,
    ).reshape(K, H)
    outs = []
    for c in range(NCH):
        g = _sc_gather(emb_pad, idx_all[c])
        outs.append(_tc_matmul(g, Wp, b))
    return jnp.concatenate(outs, axis=0)


# speedup vs baseline: 1.1448x; 1.1448x over previous
import functools

import jax
import jax.numpy as jnp
from jax import lax
from jax.experimental import pallas as pl
from jax.experimental.pallas import tpu as pltpu
from jax.experimental.pallas import tpu_sc as plsc

B = 4096
L = 50
E = 64
EP = 128
H = 512

P = 10
LP = L // P
PW = P * EP
K = L * EP

NCH = 2
BC = B // NCH

NC, NS = 2, 16
NW = NC * NS
B_PER_W = BC // NW
CHUNK = 32
ROUNDS = (B_PER_W // CHUNK) * LP

LPAD = 56

BM = 256


def _sc_gather(emb_pad, idx_c):
    mesh = plsc.VectorSubcoreMesh(core_axis_name="c", subcore_axis_name="s")

    @functools.partial(
        pl.kernel,
        mesh=mesh,
        out_type=jax.ShapeDtypeStruct((BC, K), jnp.float32),
        scratch_types=[
            pltpu.VMEM((LPAD, B_PER_W), jnp.int32),
            pltpu.VMEM((CHUNK, PW), jnp.float32),
            pltpu.VMEM((CHUNK, PW), jnp.float32),
            pltpu.SemaphoreType.DMA,
            pltpu.SemaphoreType.DMA,
            pltpu.SemaphoreType.DMA,
            pltpu.SemaphoreType.DMA,
        ],
    )
    def k(emb_hbm, idx_hbm, out_hbm, idx_v, buf0, buf1, g0, g1, w0, w1):
        wid = lax.axis_index("s") * NC + lax.axis_index("c")
        b0 = wid * B_PER_W
        pltpu.sync_copy(idx_hbm.at[pl.ds(wid * LPAD, LPAD)], idx_v)

        def fire(r, buf, sem):
            for p in range(P):
                pltpu.async_copy(
                    emb_hbm.at[idx_v.at[p * LP + r // 2, pl.ds(r % 2 * CHUNK, CHUNK)]],
                    buf.at[:, pl.ds(p * EP, EP)],
                    sem,
                )

        def out_slice(r):
            return out_hbm.at[
                pl.ds(b0 + r % 2 * CHUNK, CHUNK), pl.ds(r // 2 * PW, PW)
            ]

        def drain(buf, sem):
            pltpu.make_async_copy(
                out_hbm.at[pl.ds(0, CHUNK), pl.ds(0, PW)], buf, sem
            ).wait()

        fire(0, buf0, g0)
        fire(1, buf1, g1)

        @pl.loop(0, ROUNDS // 2)
        def _(g):
            r0 = 2 * g
            r1 = r0 + 1
            drain(buf0, g0)
            pltpu.async_copy(buf0, out_slice(r0), w0)
            drain(buf1, g1)
            pltpu.async_copy(buf1, out_slice(r1), w1)

            @pl.when(r0 + 2 < ROUNDS)
            def _():
                drain(buf0, w0)
                fire(r0 + 2, buf0, g0)

            @pl.when(r1 + 2 < ROUNDS)
            def _():
                drain(buf1, w1)
                fire(r1 + 2, buf1, g1)

        drain(buf0, w0)
        drain(buf1, w1)

    return k(emb_pad, idx_c)


def _mm_body(x_ref, w_ref, b_ref, o_ref):
    acc = jnp.dot(
        x_ref[...].astype(jnp.bfloat16),
        w_ref[...],
        preferred_element_type=jnp.float32,
    )
    o_ref[...] = jnp.maximum(acc + b_ref[...], 0.0)


def _tc_matmul(g, Wp, b):
    return pl.pallas_call(
        _mm_body,
        grid=(BC // BM,),
        in_specs=[
            pl.BlockSpec((BM, K), lambda i: (i, 0)),
            pl.BlockSpec((K, H), lambda i: (0, 0)),
            pl.BlockSpec((1, H), lambda i: (0, 0)),
        ],
        out_specs=pl.BlockSpec((BM, H), lambda i: (i, 0)),
        out_shape=jax.ShapeDtypeStruct((BC, H), jnp.float32),
    )(g, Wp, b.reshape(1, H))


def kernel(x, emb, W, b):
    idx3 = (
        x.astype(jnp.int32)
        .reshape(NCH, NW, B_PER_W, LP, P)
        .transpose(0, 1, 4, 3, 2)
        .reshape(NCH, NW, L, B_PER_W)
    )
    idx3 = jnp.pad(idx3, ((0, 0), (0, 0), (0, LPAD - L), (0, 0))).reshape(
        NCH, NW * LPAD, B_PER_W
    )
    emb_pad = jnp.pad(emb, ((0, 0), (0, EP - E)))
    Wp = jnp.concatenate(
        [
            W.reshape(L, E, H).astype(jnp.bfloat16),
            jnp.zeros((L, EP - E, H), jnp.bfloat16),
        ],
        axis=1,
    ).reshape(K, H)
    outs = []
    for c in range(NCH):
        g = _sc_gather(emb_pad, idx3[c])
        outs.append(_tc_matmul(g, Wp, b))
    return jnp.concatenate(outs, axis=0)


# speedup vs baseline: 1.2005x; 1.0486x over previous
import functools

import jax
import jax.numpy as jnp
from jax import lax
from jax.experimental import pallas as pl
from jax.experimental.pallas import tpu as pltpu
from jax.experimental.pallas import tpu_sc as plsc

B = 4096
L = 50
E = 64
EP = 128
H = 512

P = 10
LP = L // P
PW = P * EP
K = L * EP
KC = L * E

NC, NS = 2, 16
NW = NC * NS
B_PER_W = B // NW
CHUNK = 32
ROUNDS = (B_PER_W // CHUNK) * LP
IDX_PER_W = L * B_PER_W

BM = 256


def _sc_gather(emb_pad, idx_r):
    mesh = plsc.VectorSubcoreMesh(core_axis_name="c", subcore_axis_name="s")

    @functools.partial(
        pl.kernel,
        mesh=mesh,
        out_type=jax.ShapeDtypeStruct((B, K), jnp.float32),
        scratch_types=[
            pltpu.VMEM((IDX_PER_W,), jnp.int32),
            pltpu.VMEM((CHUNK, PW), jnp.float32),
            pltpu.VMEM((CHUNK, PW), jnp.float32),
            pltpu.SemaphoreType.DMA,
            pltpu.SemaphoreType.DMA,
            pltpu.SemaphoreType.DMA,
            pltpu.SemaphoreType.DMA,
        ],
    )
    def k(emb_hbm, idx_hbm, out_hbm, idx_v, buf0, buf1, g0, g1, w0, w1):
        wid = lax.axis_index("s") * NC + lax.axis_index("c")
        b0 = wid * B_PER_W
        pltpu.sync_copy(idx_hbm.at[pl.ds(wid * IDX_PER_W, IDX_PER_W)], idx_v)

        def fire(r, buf, sem):
            base = r // 4 * (4 * CHUNK) + (r % 4) * CHUNK
            for p in range(P):
                pltpu.async_copy(
                    emb_hbm.at[idx_v.at[pl.ds(p * 640 + base, CHUNK)]],
                    buf.at[:, pl.ds(p * EP, EP)],
                    sem,
                )

        def out_slice(r):
            return out_hbm.at[
                pl.ds(b0 + (r % 4) * CHUNK, CHUNK), pl.ds(r // 4 * PW, PW)
            ]

        def drain(buf, sem):
            pltpu.make_async_copy(
                out_hbm.at[pl.ds(0, CHUNK), pl.ds(0, PW)], buf, sem
            ).wait()

        fire(0, buf0, g0)
        fire(1, buf1, g1)

        @pl.loop(0, ROUNDS // 2)
        def _(g):
            r0 = 2 * g
            r1 = r0 + 1
            drain(buf0, g0)
            pltpu.async_copy(buf0, out_slice(r0), w0)
            drain(buf1, g1)
            pltpu.async_copy(buf1, out_slice(r1), w1)

            @pl.when(r0 + 2 < ROUNDS)
            def _():
                drain(buf0, w0)
                fire(r0 + 2, buf0, g0)

            @pl.when(r1 + 2 < ROUNDS)
            def _():
                drain(buf1, w1)
                fire(r1 + 2, buf1, g1)

        drain(buf0, w0)
        drain(buf1, w1)

    return k(emb_pad, idx_r)


def _mm_body(x_ref, w_ref, b_ref, o_ref):
    x = x_ref[...].astype(jnp.bfloat16).reshape(BM, L, EP)
    xc = x[:, :, :E].reshape(BM, KC)
    acc = jnp.dot(xc, w_ref[...], preferred_element_type=jnp.float32)
    o_ref[...] = jnp.maximum(acc + b_ref[...], 0.0)


def _tc_matmul(g, Wc, b):
    return pl.pallas_call(
        _mm_body,
        grid=(B // BM,),
        in_specs=[
            pl.BlockSpec((BM, K), lambda i: (i, 0)),
            pl.BlockSpec((KC, H), lambda i: (0, 0)),
            pl.BlockSpec((1, H), lambda i: (0, 0)),
        ],
        out_specs=pl.BlockSpec((BM, H), lambda i: (i, 0)),
        out_shape=jax.ShapeDtypeStruct((B, H), jnp.float32),
    )(g, Wc, b.reshape(1, H))


def kernel(x, emb, W, b):
    idx_r = (
        x.astype(jnp.int32)
        .reshape(NW, B_PER_W, LP, P)
        .transpose(0, 3, 2, 1)
        .reshape(-1)
    )
    emb_pad = jnp.pad(emb, ((0, 0), (0, EP - E)))
    Wc = W.astype(jnp.bfloat16)
    g = _sc_gather(emb_pad, idx_r)
    return _tc_matmul(g, Wc, b)


# speedup vs baseline: 1.2238x; 1.0194x over previous
import functools

import jax
import jax.numpy as jnp
from jax import lax
from jax.experimental import pallas as pl
from jax.experimental.pallas import tpu as pltpu
from jax.experimental.pallas import tpu_sc as plsc

B = 4096
L = 50
E = 64
EP = 128
H = 512

P = 10
LP = L // P
PW = P * EP
K = L * EP
KC = L * E

NC, NS = 2, 16
NW = NC * NS
B_PER_W = B // NW
CHUNK = 32
ROUNDS = (B_PER_W // CHUNK) * LP
IDX_PER_W = L * B_PER_W

BM = 512


def _sc_gather(emb_pad, idx_r):
    mesh = plsc.VectorSubcoreMesh(core_axis_name="c", subcore_axis_name="s")

    @functools.partial(
        pl.kernel,
        mesh=mesh,
        out_type=jax.ShapeDtypeStruct((B, K), jnp.float32),
        scratch_types=[
            pltpu.VMEM((IDX_PER_W,), jnp.int32),
            pltpu.VMEM((CHUNK, PW), jnp.float32),
            pltpu.VMEM((CHUNK, PW), jnp.float32),
            pltpu.SemaphoreType.DMA,
            pltpu.SemaphoreType.DMA,
            pltpu.SemaphoreType.DMA,
            pltpu.SemaphoreType.DMA,
        ],
    )
    def k(emb_hbm, idx_hbm, out_hbm, idx_v, buf0, buf1, g0, g1, w0, w1):
        wid = lax.axis_index("s") * NC + lax.axis_index("c")
        b0 = wid * B_PER_W
        pltpu.sync_copy(idx_hbm.at[pl.ds(wid * IDX_PER_W, IDX_PER_W)], idx_v)

        def fire(r, buf, sem):
            base = r // 4 * (4 * CHUNK) + (r % 4) * CHUNK
            for p in range(P):
                pltpu.async_copy(
                    emb_hbm.at[idx_v.at[pl.ds(p * 640 + base, CHUNK)]],
                    buf.at[:, pl.ds(p * EP, EP)],
                    sem,
                )

        def out_slice(r):
            return out_hbm.at[
                pl.ds(b0 + (r % 4) * CHUNK, CHUNK), pl.ds(r // 4 * PW, PW)
            ]

        def drain(buf, sem):
            pltpu.make_async_copy(
                out_hbm.at[pl.ds(0, CHUNK), pl.ds(0, PW)], buf, sem
            ).wait()

        fire(0, buf0, g0)
        fire(1, buf1, g1)

        @pl.loop(0, ROUNDS // 2)
        def _(g):
            r0 = 2 * g
            r1 = r0 + 1
            drain(buf0, g0)
            pltpu.async_copy(buf0, out_slice(r0), w0)
            drain(buf1, g1)
            pltpu.async_copy(buf1, out_slice(r1), w1)

            @pl.when(r0 + 2 < ROUNDS)
            def _():
                drain(buf0, w0)
                fire(r0 + 2, buf0, g0)

            @pl.when(r1 + 2 < ROUNDS)
            def _():
                drain(buf1, w1)
                fire(r1 + 2, buf1, g1)

        drain(buf0, w0)
        drain(buf1, w1)

    return k(emb_pad, idx_r)


def _mm_body(x_ref, w_ref, b_ref, o_ref):
    x = x_ref[...].astype(jnp.bfloat16).reshape(BM, L, EP)
    xc = x[:, :, :E].reshape(BM, KC)
    acc = jnp.dot(xc, w_ref[...], preferred_element_type=jnp.float32)
    o_ref[...] = jnp.maximum(acc + b_ref[...], 0.0)


def _tc_matmul(g, Wc, b):
    return pl.pallas_call(
        _mm_body,
        grid=(B // BM,),
        in_specs=[
            pl.BlockSpec((BM, K), lambda i: (i, 0)),
            pl.BlockSpec((KC, H), lambda i: (0, 0)),
            pl.BlockSpec((1, H), lambda i: (0, 0)),
        ],
        out_specs=pl.BlockSpec((BM, H), lambda i: (i, 0)),
        out_shape=jax.ShapeDtypeStruct((B, H), jnp.float32),
    )(g, Wc, b.reshape(1, H))


def kernel(x, emb, W, b):
    idx_r = (
        x.astype(jnp.int32)
        .reshape(NW, B_PER_W, LP, P)
        .transpose(0, 3, 2, 1)
        .reshape(-1)
    )
    emb_pad = jnp.pad(emb, ((0, 0), (0, EP - E)))
    Wc = W.astype(jnp.bfloat16)
    g = _sc_gather(emb_pad, idx_r)
    return _tc_matmul(g, Wc, b)
